# Initial kernel scaffold; baseline (speedup 1.0000x reference)
import jax
import jax.numpy as jnp
from jax.experimental import pallas as pl


def kernel(atomic_number, edge_index, pos, embed, Wf, Wphi, bphi, W1, b1, W2, b2):
    raise NotImplementedError("write your pallas kernel here")


# speedup vs baseline: 1.3158x; 1.3158x over previous
import functools
import jax
import jax.numpy as jnp
import numpy as np
from jax import lax
from jax.experimental import pallas as pl
from jax.experimental.pallas import tpu as pltpu
from jax.experimental.pallas import tpu_sc as plsc

N = 10000
E = 160000
F = 128
T = 4
K = 8
NL = 4
RC = 5.0

N_PAD = 10240
E_PAD = 163840
CHUNK = 128
F32 = jnp.float32

_NC = 2
_NS = 16
_NW = _NC * _NS


@functools.lru_cache(maxsize=None)
def _make_gather(V, D):
    per_w = E_PAD // _NW
    n_chunks = per_w // CHUNK
    mesh = plsc.VectorSubcoreMesh(core_axis_name="c", subcore_axis_name="s", num_cores=_NC, num_subcores=_NS)

    @functools.partial(
        pl.kernel, mesh=mesh,
        out_type=jax.ShapeDtypeStruct((E_PAD, D), F32),
        scratch_types=[
            pltpu.VMEM((CHUNK,), jnp.int32),
            pltpu.VMEM((CHUNK, D), F32),
            pltpu.SemaphoreType.DMA,
        ],
    )
    def k(table_hbm, idx_hbm, out_hbm, idx_v, rows_v, sem):
        wid = lax.axis_index("s") * _NC + lax.axis_index("c")
        base = wid * per_w

        def body(j, carry):
            off = base + j * CHUNK
            pltpu.sync_copy(idx_hbm.at[pl.ds(off, CHUNK)], idx_v)
            pltpu.async_copy(table_hbm.at[idx_v], rows_v, sem).wait()
            pltpu.sync_copy(rows_v, out_hbm.at[pl.ds(off, CHUNK), :])
            return carry

        lax.fori_loop(0, n_chunks, body, 0)

    return k


@functools.lru_cache(maxsize=None)
def _make_scatter_add():
    per_s = E_PAD // _NS
    n_chunks = per_s // CHUNK
    stripe = N_PAD // _NS
    mesh = plsc.VectorSubcoreMesh(core_axis_name="c", subcore_axis_name="s", num_cores=_NC, num_subcores=_NS)

    @functools.partial(
        pl.kernel, mesh=mesh,
        out_type=(jax.ShapeDtypeStruct((N_PAD, F), F32),
                  jax.ShapeDtypeStruct((N_PAD, F), F32)),
        scratch_types=[
            pltpu.VMEM((CHUNK,), jnp.int32),
            pltpu.VMEM((CHUNK, F), F32),
            pltpu.VMEM_SHARED((N_PAD, F), F32),
        ],
    )
    def k(dst_hbm, a_hbm, b_hbm, zeros_hbm, a_out, b_out, idx_v, rows_v, acc):
        cid = lax.axis_index("c")
        sid = lax.axis_index("s")
        pltpu.sync_copy(zeros_hbm, acc.at[pl.ds(sid * stripe, stripe), :])
        plsc.subcore_barrier()

        def stream(src_ref):
            def body(j, carry):
                off = sid * per_s + j * CHUNK
                pltpu.sync_copy(dst_hbm.at[pl.ds(off, CHUNK)], idx_v)
                pltpu.sync_copy(src_ref.at[pl.ds(off, CHUNK), :], rows_v)
                pltpu.sync_copy(rows_v, acc.at[idx_v], add=True)
                return carry
            lax.fori_loop(0, n_chunks, body, 0)

        @pl.when(cid == 0)
        def _():
            stream(a_hbm)

        @pl.when(cid == 1)
        def _():
            stream(b_hbm)

        plsc.subcore_barrier()

        @pl.when(cid == 0)
        def _():
            pltpu.sync_copy(acc.at[pl.ds(sid * stripe, stripe), :],
                            a_out.at[pl.ds(sid * stripe, stripe), :])

        @pl.when(cid == 1)
        def _():
            pltpu.sync_copy(acc.at[pl.ds(sid * stripe, stripe), :],
                            b_out.at[pl.ds(sid * stripe, stripe), :])

    return k


_BLK = 512


def _embed_body(oh_ref, emb_ref, out_ref):
    out_ref[...] = jnp.dot(oh_ref[...], emb_ref[...],
                           preferred_element_type=F32)


def _embed_call(onehot, embed_pad):
    return pl.pallas_call(
        _embed_body,
        grid=(N_PAD // _BLK,),
        in_specs=[pl.BlockSpec((_BLK, 8), lambda i: (i, 0)),
                  pl.BlockSpec((8, F), lambda i: (0, 0))],
        out_specs=pl.BlockSpec((_BLK, F), lambda i: (i, 0)),
        out_shape=jax.ShapeDtypeStruct((N_PAD, F), F32),
    )(onehot, embed_pad)


def _geom_body(ps_ref, pd_ref, tsel_ref, r_ref, dir_ref):
    diff = ps_ref[...][:, :16] - pd_ref[...][:, :16]
    d2 = jnp.sum(diff * diff, axis=1, keepdims=True) + 1e-8
    d = jnp.sqrt(d2)
    dir_ref[...] = diff / d
    fc = 0.5 * (jnp.cos(jnp.pi * jnp.clip(d, 0.0, RC) / RC) + 1.0)
    kv = (lax.broadcasted_iota(jnp.int32, (_BLK, T * K), 1) % K + 1).astype(F32)
    rbf = jnp.sin(kv * (jnp.pi / RC) * d) / d
    r_ref[...] = rbf * fc * tsel_ref[...]


def _geom_call(ps, pdst, tsel):
    return pl.pallas_call(
        _geom_body,
        grid=(E_PAD // _BLK,),
        in_specs=[pl.BlockSpec((_BLK, 128), lambda i: (i, 0)),
                  pl.BlockSpec((_BLK, 128), lambda i: (i, 0)),
                  pl.BlockSpec((_BLK, T * K), lambda i: (i, 0))],
        out_specs=[pl.BlockSpec((_BLK, T * K), lambda i: (i, 0)),
                   pl.BlockSpec((_BLK, 16), lambda i: (i, 0))],
        out_shape=[jax.ShapeDtypeStruct((E_PAD, T * K), F32),
                   jax.ShapeDtypeStruct((E_PAD, 16), F32)],
    )(ps, pdst, tsel)


def _phi_body(s_ref, w_ref, b_ref, out_ref):
    out_ref[0] = jnp.dot(s_ref[...], w_ref[0],
                         preferred_element_type=F32) + b_ref[0]


def _phi_call(s, Wphi_l, bphi_l):
    return pl.pallas_call(
        _phi_body,
        grid=(T, N_PAD // _BLK),
        in_specs=[pl.BlockSpec((_BLK, F), lambda t, i: (i, 0)),
                  pl.BlockSpec((1, F, 3 * F), lambda t, i: (t, 0, 0)),
                  pl.BlockSpec((1, 1, 3 * F), lambda t, i: (t, 0, 0))],
        out_specs=pl.BlockSpec((1, _BLK, 3 * F), lambda t, i: (t, i, 0)),
        out_shape=jax.ShapeDtypeStruct((T, N_PAD, 3 * F), F32),
    )(s, Wphi_l, bphi_l.reshape(T, 1, 3 * F))


def _msg_body(r_ref, wf_ref, phis_ref, dir_ref, nv_ref, ms_ref, m0_ref,
              m1_ref, m2_ref, has_nv):
    w = jnp.dot(r_ref[...], wf_ref[...],
                preferred_element_type=F32) * (1.0 / T)
    phis = phis_ref[...]
    gs = phis[:, :F]
    gv = phis[:, F:2 * F]
    gd = phis[:, 2 * F:]
    ms_ref[...] = gs * w
    a = gv * w
    b = gd * w
    dirv = dir_ref[...]
    outs = (m0_ref, m1_ref, m2_ref)
    for c in range(3):
        if has_nv:
            nvc = nv_ref[:, c * F:(c + 1) * F]
            outs[c][...] = nvc * a + b * dirv[:, c:c + 1]
        else:
            outs[c][...] = b * dirv[:, c:c + 1]


def _msg_call(Rb, Wfl, phis, dirv, nvsrc):
    has_nv = nvsrc is not None
    in_specs = [pl.BlockSpec((_BLK, T * K), lambda i: (i, 0)),
                pl.BlockSpec((T * K, F), lambda i: (0, 0)),
                pl.BlockSpec((_BLK, 3 * F), lambda i: (i, 0)),
                pl.BlockSpec((_BLK, 16), lambda i: (i, 0))]
    args = [Rb, Wfl, phis, dirv]
    if has_nv:
        in_specs.append(pl.BlockSpec((_BLK, 3 * F), lambda i: (i, 0)))
        args.append(nvsrc)
        body = functools.partial(_msg_body, has_nv=True)
    else:
        def body(r, wf, ph, di, ms, m0, m1, m2):
            _msg_body(r, wf, ph, di, None, ms, m0, m1, m2, has_nv=False)
    return pl.pallas_call(
        body,
        grid=(E_PAD // _BLK,),
        in_specs=in_specs,
        out_specs=[pl.BlockSpec((_BLK, F), lambda i: (i, 0))] * 4,
        out_shape=[jax.ShapeDtypeStruct((E_PAD, F), F32)] * 4,
    )(*args)


def _upd_body(s_ref, ss_ref, v_ref, v0_ref, v1_ref, v2_ref, so_ref, vo_ref):
    so_ref[...] = s_ref[...] + ss_ref[...]
    vo_ref[...] = v_ref[...] + jnp.concatenate(
        [v0_ref[...], v1_ref[...], v2_ref[...]], axis=1)


def _upd_call(s, ssum, vflat, v0, v1, v2):
    return pl.pallas_call(
        _upd_body,
        grid=(N_PAD // _BLK,),
        in_specs=[pl.BlockSpec((_BLK, F), lambda i: (i, 0)),
                  pl.BlockSpec((_BLK, F), lambda i: (i, 0)),
                  pl.BlockSpec((_BLK, 3 * F), lambda i: (i, 0)),
                  pl.BlockSpec((_BLK, F), lambda i: (i, 0)),
                  pl.BlockSpec((_BLK, F), lambda i: (i, 0)),
                  pl.BlockSpec((_BLK, F), lambda i: (i, 0))],
        out_specs=[pl.BlockSpec((_BLK, F), lambda i: (i, 0)),
                   pl.BlockSpec((_BLK, 3 * F), lambda i: (i, 0))],
        out_shape=[jax.ShapeDtypeStruct((N_PAD, F), F32),
                   jax.ShapeDtypeStruct((N_PAD, 3 * F), F32)],
    )(s, ssum, vflat, v0, v1, v2)


def _final_body(s_ref, w1_ref, b1_ref, w2_ref, b2_ref, out_ref):
    pooled = jnp.sum(s_ref[...], axis=0, keepdims=True)
    h = jnp.dot(pooled, w1_ref[...], preferred_element_type=F32) + b1_ref[...]
    sp = jnp.maximum(h, 0.0) + jnp.log(1.0 + jnp.exp(-jnp.abs(h)))
    h = sp - np.log(2.0)
    out_ref[...] = jnp.dot(h, w2_ref[...], preferred_element_type=F32) + b2_ref[...]


def _final_call(s, W1, b1, W2, b2):
    return pl.pallas_call(
        _final_body,
        out_shape=jax.ShapeDtypeStruct((1, 1), F32),
    )(s, W1, b1.reshape(1, F), W2, b2.reshape(1, 1))


def kernel(atomic_number, edge_index, pos, embed, Wf, Wphi, bphi, W1, b1, W2, b2):
    _gather_phi = _make_gather(T * N_PAD, 3 * F)
    _gather_nv = _make_gather(N_PAD, 3 * F)
    _gather_pos = _make_gather(N_PAD, 128)
    _scatter2 = _make_scatter_add()
    src = edge_index[0].astype(jnp.int32)
    dst = edge_index[1].astype(jnp.int32)
    atom = atomic_number.astype(jnp.int32)
    t_e = atom[dst]

    pad_e = E_PAD - E
    src_pad = jnp.pad(src, (0, pad_e))
    dst_pad = jnp.pad(dst, (0, pad_e), constant_values=N)
    gidx = jnp.pad(t_e * N_PAD + src, (0, pad_e))
    tsel = (jnp.arange(T * K, dtype=jnp.int32)[None, :] // K
            == t_e[:, None]).astype(F32)
    tsel = jnp.pad(tsel, ((0, pad_e), (0, 0)))

    pos_pad = jnp.zeros((N_PAD, 128), F32).at[:N, :3].set(pos.astype(F32))
    onehot = (jnp.arange(8, dtype=jnp.int32)[None, :]
              == atom[:, None]).astype(F32)
    onehot = jnp.pad(onehot, ((0, N_PAD - N), (0, 0)))
    embed_pad = jnp.pad(embed.astype(F32), ((0, 8 - T), (0, 0)))
    zeros_stripe = jnp.zeros((N_PAD // _NS, F), F32)
    Wf_flat = Wf.astype(F32).reshape(NL, T * K, F)

    ps = _gather_pos(pos_pad, src_pad)
    pdst = _gather_pos(pos_pad, dst_pad)
    Rb, dirv = _geom_call(ps, pdst, tsel)

    s = _embed_call(onehot, embed_pad)
    vflat = jnp.zeros((N_PAD, 3 * F), F32)

    for l in range(NL):
        phi_st = _phi_call(s, Wphi[l].astype(F32), bphi[l].astype(F32))
        phis = _gather_phi(phi_st.reshape(T * N_PAD, 3 * F), gidx)
        nvsrc = _gather_nv(vflat, src_pad) if l > 0 else None
        ms, m0, m1, m2 = _msg_call(Rb, Wf_flat[l], phis, dirv, nvsrc)
        ssum, v0 = _scatter2(dst_pad, ms, m0, zeros_stripe)
        v1, v2 = _scatter2(dst_pad, m1, m2, zeros_stripe)
        s, vflat = _upd_call(s, ssum, vflat, v0, v1, v2)

    return _final_call(s, W1.astype(F32), b1.astype(F32),
                       W2.astype(F32), b2.astype(F32))
